# baseline (speedup 1.0000x reference)
import functools

import jax
import jax.numpy as jnp
from jax import lax
from jax.experimental import pallas as pl
from jax.experimental.pallas import tpu as pltpu
from jax.experimental.pallas import tpu_sc as plsc

F32 = jnp.float32
BF16 = jnp.bfloat16
I32 = jnp.int32

NUM_TILES = 32


def _tile_id():
    return lax.axis_index("s") * 2 + lax.axis_index("c")


def _router_body(cap, capp, x_ref, wg_ref, xb_ref, dest_ref, islot_ref,
                 wval_ref, laux_ref, lload_ref):
    T, D = x_ref.shape
    E = wg_ref.shape[0]
    CH = 1024
    x = x_ref[...]
    xb_ref[...] = x.astype(BF16)
    logits = lax.dot_general(x, wg_ref[...], (((1,), (1,)), ((), ())),
                             preferred_element_type=F32)
    m = jnp.max(logits, axis=1, keepdims=True)
    ex = jnp.exp(logits - m)
    gates = ex / jnp.sum(ex, axis=1, keepdims=True)

    it8 = lax.broadcasted_iota(I32, (T, E), 1)
    g1 = jnp.max(gates, axis=1, keepdims=True)
    i1 = jnp.min(jnp.where(gates == g1, it8, E), axis=1, keepdims=True)
    gm = jnp.where(it8 == i1, -jnp.inf, gates)
    g2 = jnp.max(gm, axis=1, keepdims=True)
    i2 = jnp.min(jnp.where(gm == g2, it8, E), axis=1, keepdims=True)

    imp = jnp.sum(gates, axis=0, keepdims=True)
    mi = jnp.mean(imp)
    si = jnp.sqrt(jnp.mean((imp - mi) ** 2))
    imp_loss = (si / (mi + 1e-6)) ** 2

    r = lax.broadcasted_iota(I32, (CH, CH), 0)
    c = lax.broadcasted_iota(I32, (CH, CH), 1)
    tri = (r >= c).astype(F32)
    ohA = (i1 == it8).astype(F32)
    ohB = (i2 == it8).astype(F32)
    carry = jnp.zeros((1, E), F32)
    pos = {0: [], 1: []}
    for key, oh in ((0, ohA), (1, ohB)):
        for ci in range(T // CH):
            blk = lax.slice(oh, (ci * CH, 0), ((ci + 1) * CH, E))
            cs = lax.dot_general(tri, blk, (((1,), (0,)), ((), ())),
                                 preferred_element_type=F32) + carry
            pos[key].append(jnp.sum(cs * blk, axis=1, keepdims=True) - 1.0)
            carry = carry + jnp.sum(blk, axis=0, keepdims=True)
    posA = jnp.concatenate(pos[0], axis=0)
    posB = jnp.concatenate(pos[1], axis=0)

    tpe = jnp.minimum(carry, float(cap))
    mt = jnp.mean(tpe)
    st = jnp.sqrt(jnp.mean((tpe - mt) ** 2))
    l_load = (st / (mt + 1e-6)) ** 2
    laux_ref[...] = jnp.reshape(0.5 * (imp_loss + l_load), (1, 1))
    lload_ref[...] = jnp.reshape(l_load, (1, 1))

    trash = cap
    dropg = cap + 1
    for i_, po, g_, col in ((i1, posA, g1, 0), (i2, posB, g2, 1)):
        pin = po.astype(I32)
        within = po < float(cap)
        slot = i_ * capp + pin
        dest_ref[:, col:col + 1] = jnp.where(within, slot, trash)
        islot_ref[:, col:col + 1] = jnp.where(within, slot, dropg)
        wval_ref[:, col:col + 1] = g_


def _make_scatter(nslot, n):
    mesh = plsc.VectorSubcoreMesh(core_axis_name="c", subcore_axis_name="s")

    @functools.partial(
        pl.kernel,
        out_type=(jax.ShapeDtypeStruct((nslot,), I32),
                  jax.ShapeDtypeStruct((nslot,), F32)),
        mesh=mesh,
        scratch_types=[
            pltpu.VMEM((n,), I32),
            pltpu.VMEM((n,), I32),
            pltpu.VMEM((n,), F32),
            pltpu.VMEM((nslot,), I32),
            pltpu.VMEM((nslot,), F32),
        ],
        compiler_params=pltpu.CompilerParams(needs_layout_passes=False),
    )
    def scat(dest_hbm, tok_hbm, wv_hbm, src_hbm, wslot_hbm,
             destv, tokv, wvv, srcv, wsv):
        @pl.when(_tile_id() == 0)
        def _():
            pltpu.sync_copy(dest_hbm, destv)
            pltpu.sync_copy(tok_hbm, tokv)
            pltpu.sync_copy(wv_hbm, wvv)
            zi = jnp.zeros((16,), I32)
            zf = jnp.zeros((16,), F32)

            def zb(i, _):
                srcv[pl.ds(i * 16, 16)] = zi
                wsv[pl.ds(i * 16, 16)] = zf
                return 0

            lax.fori_loop(0, nslot // 16, zb, 0)

            def sb(i, _):
                idx = destv[pl.ds(i * 16, 16)]
                plsc.store_scatter(srcv, [idx], tokv[pl.ds(i * 16, 16)])
                plsc.store_scatter(wsv, [idx], wvv[pl.ds(i * 16, 16)])
                return 0

            lax.fori_loop(0, n // 16, sb, 0)
            pltpu.sync_copy(srcv, src_hbm)
            pltpu.sync_copy(wsv, wslot_hbm)

    return scat


def _make_dispatch(nslot, D):
    per = nslot // NUM_TILES
    mesh = plsc.VectorSubcoreMesh(core_axis_name="c", subcore_axis_name="s")
    chunks = []
    off = 0
    while off < per:
        sz = min(64, per - off)
        chunks.append((off, sz))
        off += sz

    @functools.partial(
        pl.kernel,
        out_type=jax.ShapeDtypeStruct((nslot, D), I32),
        mesh=mesh,
        scratch_types=[
            pltpu.VMEM((per,), I32),
            pltpu.VMEM((64, D), I32),
            pltpu.SemaphoreType.DMA,
        ],
    )
    def disp(src_hbm, x_hbm, xd_hbm, idxv, rows, sem):
        base = _tile_id() * per
        pltpu.sync_copy(src_hbm.at[pl.ds(base, per)], idxv)
        for off, sz in chunks:
            pltpu.async_copy(x_hbm.at[idxv.at[pl.ds(off, sz)]],
                             rows.at[pl.ds(0, sz)], sem).wait()
            pltpu.sync_copy(rows.at[pl.ds(0, sz)],
                            xd_hbm.at[pl.ds(base + off, sz)])

    return disp


def _ffn_body(nh, xd_ref, w1_ref, b1_ref, w2_ref, b2_ref, wc_ref, y_ref,
              w1b, w2b, gbuf, accum):
    s = pl.program_id(0)
    p = lax.rem(s, 2)
    q = lax.rem(s + 1, 2)

    w1b[p] = w1_ref[0].astype(BF16)
    w2b[p] = w2_ref[0].astype(BF16)

    a = lax.dot_general(xd_ref[0], w1b[q], (((1,), (0,)), ((), ())),
                        preferred_element_type=F32)
    a = a + b1_ref[0]
    g = 0.5 * a * (1.0 + lax.erf(a * 0.7071067811865476))
    gbuf[p] = g.astype(BF16)

    accum[...] = accum[...] + lax.dot_general(
        gbuf[q], w2b[q], (((1,), (0,)), ((), ())),
        preferred_element_type=F32)

    @pl.when(s == 1)
    def _():
        accum[...] = jnp.zeros_like(accum)

    @pl.when((s >= 2) & (lax.rem(s - 2, nh) == nh - 1))
    def _():
        y_ref[0] = (accum[...] + b2_ref[0]) * wc_ref[0]
        accum[...] = jnp.zeros_like(accum)


def _make_combine(T, D):
    per = T // NUM_TILES
    cs = 32
    mesh = plsc.VectorSubcoreMesh(core_axis_name="c", subcore_axis_name="s")

    @functools.partial(
        pl.kernel,
        out_type=jax.ShapeDtypeStruct((T, D), F32),
        mesh=mesh,
        scratch_types=[
            pltpu.VMEM((per,), I32),
            pltpu.VMEM((per,), I32),
            pltpu.VMEM((cs, D), F32),
            pltpu.VMEM((cs, D), F32),
            pltpu.SemaphoreType.DMA,
            pltpu.SemaphoreType.DMA,
        ],
    )
    def comb(ys_hbm, i0_hbm, i1_hbm, out_hbm, i0v, i1v, b0, b1, s0, s1):
        base = _tile_id() * per
        pltpu.sync_copy(i0_hbm.at[pl.ds(base, per)], i0v)
        pltpu.sync_copy(i1_hbm.at[pl.ds(base, per)], i1v)
        ngrp = D // 16
        for ci in range(per // cs):
            off = ci * cs
            cp0 = pltpu.async_copy(ys_hbm.at[i0v.at[pl.ds(off, cs)]], b0, s0)
            cp1 = pltpu.async_copy(ys_hbm.at[i1v.at[pl.ds(off, cs)]], b1, s1)
            cp0.wait()
            cp1.wait()

            def add_row(rr, _):
                for u in range(ngrp):
                    b0[rr, pl.ds(u * 16, 16)] = (
                        b0[rr, pl.ds(u * 16, 16)] + b1[rr, pl.ds(u * 16, 16)])
                return 0

            lax.fori_loop(0, cs, add_row, 0)
            pltpu.sync_copy(b0, out_hbm.at[pl.ds(base + off, cs)])

    return comb


def kernel(x, Wg, W1, b1, W2, b2):
    bsz, seq, D = x.shape
    E, _, H = W1.shape
    T = bsz * seq
    K = 2
    cap = int(round(K * T * 1.05 / E))
    capp = -((-(cap + 2)) // 64) * 64
    nslot = E * capp
    nh = 8

    x2d = x.reshape(T, D)

    router = pl.pallas_call(
        functools.partial(_router_body, cap, capp),
        out_shape=(
            jax.ShapeDtypeStruct((T, D), BF16),
            jax.ShapeDtypeStruct((T, 2), I32),
            jax.ShapeDtypeStruct((T, 2), I32),
            jax.ShapeDtypeStruct((T, 2), F32),
            jax.ShapeDtypeStruct((1, 1), F32),
            jax.ShapeDtypeStruct((1, 1), F32),
        ),
    )
    xb, dest4, islot4, wval4, laux, lload = router(x2d, Wg)

    dest_sl = jnp.concatenate([dest4[:, 0], dest4[:, 1]])
    wval_sl = jnp.concatenate([wval4[:, 0], wval4[:, 1]])
    tok = jnp.arange(T, dtype=I32)
    tok_sl = jnp.concatenate([tok, tok])

    src, wslot = _make_scatter(nslot, K * T)(dest_sl, tok_sl, wval_sl)
    xb32 = lax.bitcast_convert_type(xb.reshape(T, D // 2, 2), I32)
    xd32 = _make_dispatch(nslot, D // 2)(src, xb32)
    xd = lax.bitcast_convert_type(xd32, BF16).reshape(nslot, D)

    hb = H // nh
    last = E * nh

    def _cur(s):
        return jnp.minimum(s, last - 1)

    def _prev(s):
        return jnp.clip(s - 1, 0, last - 1)

    def _out_e(s):
        return jnp.maximum(s - (nh + 1), 0) // nh

    ffn = pl.pallas_call(
        functools.partial(_ffn_body, nh),
        grid=(last + 2,),
        in_specs=[
            pl.BlockSpec((1, capp, D), lambda s: (_prev(s) // nh, 0, 0)),
            pl.BlockSpec((1, D, hb),
                         lambda s: (_cur(s) // nh, 0, _cur(s) % nh)),
            pl.BlockSpec((1, 1, hb),
                         lambda s: (_prev(s) // nh, 0, _prev(s) % nh)),
            pl.BlockSpec((1, hb, D),
                         lambda s: (_prev(s) // nh, _prev(s) % nh, 0)),
            pl.BlockSpec((1, 1, D), lambda s: (_out_e(s), 0, 0)),
            pl.BlockSpec((1, capp, 1), lambda s: (_out_e(s), 0, 0)),
        ],
        out_specs=pl.BlockSpec((1, capp, D), lambda s: (_out_e(s), 0, 0)),
        out_shape=jax.ShapeDtypeStruct((E, capp, D), F32),
        scratch_shapes=[
            pltpu.VMEM((2, D, hb), BF16),
            pltpu.VMEM((2, hb, D), BF16),
            pltpu.VMEM((2, capp, hb), BF16),
            pltpu.VMEM((capp, D), F32),
        ],
        compiler_params=pltpu.CompilerParams(
            dimension_semantics=("arbitrary",)),
    )
    y = ffn(xd.reshape(E, capp, D), W1, b1.reshape(E, 1, H), W2,
            b2.reshape(E, 1, D), wslot.reshape(E, capp, 1))

    out2d = _make_combine(T, D)(y.reshape(nslot, D),
                                islot4[:, 0], islot4[:, 1])
    return (out2d.reshape(bsz, seq, D), laux[0, 0], lload[0, 0])


# speedup vs baseline: 1.5878x; 1.5878x over previous
import functools

import jax
import jax.numpy as jnp
from jax import lax
from jax.experimental import pallas as pl
from jax.experimental.pallas import tpu as pltpu
from jax.experimental.pallas import tpu_sc as plsc

F32 = jnp.float32
BF16 = jnp.bfloat16
I32 = jnp.int32

NUM_TILES = 32


def _tile_id():
    return lax.axis_index("s") * 2 + lax.axis_index("c")


def _router_body(cap, capp, x_ref, wg_ref, dest_ref, islot_ref,
                 wval_ref, laux_ref, lload_ref):
    T, D = x_ref.shape
    E = wg_ref.shape[0]
    CH = 1024
    x = x_ref[...]
    logits = lax.dot_general(x, wg_ref[...], (((1,), (1,)), ((), ())),
                             preferred_element_type=F32)
    m = jnp.max(logits, axis=1, keepdims=True)
    ex = jnp.exp(logits - m)
    gates = ex / jnp.sum(ex, axis=1, keepdims=True)

    it8 = lax.broadcasted_iota(I32, (T, E), 1)
    g1 = jnp.max(gates, axis=1, keepdims=True)
    i1 = jnp.min(jnp.where(gates == g1, it8, E), axis=1, keepdims=True)
    gm = jnp.where(it8 == i1, -jnp.inf, gates)
    g2 = jnp.max(gm, axis=1, keepdims=True)
    i2 = jnp.min(jnp.where(gm == g2, it8, E), axis=1, keepdims=True)

    imp = jnp.sum(gates, axis=0, keepdims=True)
    mi = jnp.mean(imp)
    si = jnp.sqrt(jnp.mean((imp - mi) ** 2))
    imp_loss = (si / (mi + 1e-6)) ** 2

    r = lax.broadcasted_iota(I32, (CH, CH), 0)
    c = lax.broadcasted_iota(I32, (CH, CH), 1)
    tri = (r >= c).astype(F32)
    ohA = (i1 == it8).astype(F32)
    ohB = (i2 == it8).astype(F32)
    carry = jnp.zeros((1, E), F32)
    pos = {0: [], 1: []}
    for key, oh in ((0, ohA), (1, ohB)):
        for ci in range(T // CH):
            blk = lax.slice(oh, (ci * CH, 0), ((ci + 1) * CH, E))
            cs = lax.dot_general(tri, blk, (((1,), (0,)), ((), ())),
                                 preferred_element_type=F32) + carry
            pos[key].append(jnp.sum(cs * blk, axis=1, keepdims=True) - 1.0)
            carry = carry + jnp.sum(blk, axis=0, keepdims=True)
    posA = jnp.concatenate(pos[0], axis=0)
    posB = jnp.concatenate(pos[1], axis=0)

    tpe = jnp.minimum(carry, float(cap))
    mt = jnp.mean(tpe)
    st = jnp.sqrt(jnp.mean((tpe - mt) ** 2))
    l_load = (st / (mt + 1e-6)) ** 2
    laux_ref[...] = jnp.reshape(0.5 * (imp_loss + l_load), (1, 1))
    lload_ref[...] = jnp.reshape(l_load, (1, 1))

    trash = cap
    dropg = cap + 1
    for i_, po, g_, col in ((i1, posA, g1, 0), (i2, posB, g2, 1)):
        pin = po.astype(I32)
        within = po < float(cap)
        slot = i_ * capp + pin
        dest_ref[:, col:col + 1] = jnp.where(within, slot, trash)
        islot_ref[:, col:col + 1] = jnp.where(within, slot, dropg)
        wval_ref[:, col:col + 1] = g_


def _make_scatter(nslot, n):
    mesh = plsc.VectorSubcoreMesh(core_axis_name="c", subcore_axis_name="s")

    @functools.partial(
        pl.kernel,
        out_type=(jax.ShapeDtypeStruct((nslot,), I32),
                  jax.ShapeDtypeStruct((nslot,), F32)),
        mesh=mesh,
        scratch_types=[
            pltpu.VMEM((n,), I32),
            pltpu.VMEM((n,), I32),
            pltpu.VMEM((n,), F32),
            pltpu.VMEM((nslot,), I32),
            pltpu.VMEM((nslot,), F32),
        ],
        compiler_params=pltpu.CompilerParams(needs_layout_passes=False),
    )
    def scat(dest_hbm, tok_hbm, wv_hbm, src_hbm, wslot_hbm,
             destv, tokv, wvv, srcv, wsv):
        @pl.when(_tile_id() == 0)
        def _():
            pltpu.sync_copy(dest_hbm, destv)
            pltpu.sync_copy(tok_hbm, tokv)
            pltpu.sync_copy(wv_hbm, wvv)
            zi = jnp.zeros((16,), I32)
            zf = jnp.zeros((16,), F32)

            def zb(i, _):
                srcv[pl.ds(i * 16, 16)] = zi
                wsv[pl.ds(i * 16, 16)] = zf
                return 0

            lax.fori_loop(0, nslot // 16, zb, 0)

            def sb(i, _):
                idx = destv[pl.ds(i * 16, 16)]
                plsc.store_scatter(srcv, [idx], tokv[pl.ds(i * 16, 16)])
                plsc.store_scatter(wsv, [idx], wvv[pl.ds(i * 16, 16)])
                return 0

            lax.fori_loop(0, n // 16, sb, 0)
            pltpu.sync_copy(srcv, src_hbm)
            pltpu.sync_copy(wsv, wslot_hbm)

    return scat


def _make_dispatch(nslot, D):
    per = nslot // NUM_TILES
    mesh = plsc.VectorSubcoreMesh(core_axis_name="c", subcore_axis_name="s")
    chunks = []
    off = 0
    while off < per:
        sz = min(32, per - off)
        chunks.append((off, sz))
        off += sz

    @functools.partial(
        pl.kernel,
        out_type=jax.ShapeDtypeStruct((nslot, D), F32),
        mesh=mesh,
        scratch_types=[
            pltpu.VMEM((per,), I32),
            pltpu.VMEM((32, D), F32),
            pltpu.VMEM((32, D), F32),
            pltpu.SemaphoreType.DMA,
            pltpu.SemaphoreType.DMA,
        ],
    )
    def disp(src_hbm, x_hbm, xd_hbm, idxv, rows0, rows1, sem0, sem1):
        base = _tile_id() * per
        pltpu.sync_copy(src_hbm.at[pl.ds(base, per)], idxv)
        bufs = (rows0, rows1)
        sems = (sem0, sem1)
        n = len(chunks)
        cps = [None] * n

        def issue(i):
            off, sz = chunks[i]
            cps[i] = pltpu.async_copy(x_hbm.at[idxv.at[pl.ds(off, sz)]],
                                      bufs[i % 2].at[pl.ds(0, sz)],
                                      sems[i % 2])

        issue(0)
        for i, (off, sz) in enumerate(chunks):
            if i + 1 < n:
                issue(i + 1)
            cps[i].wait()
            pltpu.sync_copy(bufs[i % 2].at[pl.ds(0, sz)],
                            xd_hbm.at[pl.ds(base + off, sz)])

    return disp


def _ffn_body(nh, xd_ref, w1_ref, b1_ref, w2_ref, b2_ref, wc_ref, y_ref,
              xbf):
    h = pl.program_id(1)

    @pl.when(h == 0)
    def _():
        xbf[...] = xd_ref[0].astype(BF16)
        y_ref[...] = jnp.zeros_like(y_ref)

    a = lax.dot_general(xbf[...], w1_ref[0].astype(BF16),
                        (((1,), (0,)), ((), ())), preferred_element_type=F32)
    a = a + b1_ref[0]
    g = 0.5 * a * (1.0 + lax.erf(a * 0.7071067811865476))
    acc = lax.dot_general(g.astype(BF16), w2_ref[0].astype(BF16),
                          (((1,), (0,)), ((), ())), preferred_element_type=F32)
    ynew = y_ref[0] + acc

    @pl.when(h < nh - 1)
    def _():
        y_ref[0] = ynew

    @pl.when(h == nh - 1)
    def _():
        y_ref[0] = (ynew + b2_ref[0]) * wc_ref[0]


def _make_combine(T, D):
    per = T // NUM_TILES
    cs = 8
    nch = per // cs
    mesh = plsc.VectorSubcoreMesh(core_axis_name="c", subcore_axis_name="s")

    @functools.partial(
        pl.kernel,
        out_type=jax.ShapeDtypeStruct((T, D), F32),
        mesh=mesh,
        scratch_types=[
            pltpu.VMEM((per,), I32),
            pltpu.VMEM((per,), I32),
            pltpu.VMEM((cs, D), F32),
            pltpu.VMEM((cs, D), F32),
            pltpu.VMEM((cs, D), F32),
            pltpu.VMEM((cs, D), F32),
            pltpu.SemaphoreType.DMA,
            pltpu.SemaphoreType.DMA,
            pltpu.SemaphoreType.DMA,
            pltpu.SemaphoreType.DMA,
        ],
    )
    def comb(ys_hbm, i0_hbm, i1_hbm, out_hbm, i0v, i1v,
             a0, a1, c0, c1, sa0, sa1, sc0, sc1):
        base = _tile_id() * per
        pltpu.sync_copy(i0_hbm.at[pl.ds(base, per)], i0v)
        pltpu.sync_copy(i1_hbm.at[pl.ds(base, per)], i1v)
        ngrp = D // 16
        abufs = (a0, a1)
        cbufs = (c0, c1)
        asems = (sa0, sa1)
        csems = (sc0, sc1)
        cpa = [None] * nch
        cpc = [None] * nch

        def issue(i):
            off = i * cs
            cpa[i] = pltpu.async_copy(ys_hbm.at[i0v.at[pl.ds(off, cs)]],
                                      abufs[i % 2], asems[i % 2])
            cpc[i] = pltpu.async_copy(ys_hbm.at[i1v.at[pl.ds(off, cs)]],
                                      cbufs[i % 2], csems[i % 2])

        issue(0)
        for ci in range(nch):
            if ci + 1 < nch:
                issue(ci + 1)
            cpa[ci].wait()
            cpc[ci].wait()
            aa = abufs[ci % 2]
            cc = cbufs[ci % 2]

            def add_row(rr, _):
                for u in range(ngrp):
                    aa[rr, pl.ds(u * 16, 16)] = (
                        aa[rr, pl.ds(u * 16, 16)] + cc[rr, pl.ds(u * 16, 16)])
                return 0

            lax.fori_loop(0, cs, add_row, 0)
            pltpu.sync_copy(aa, out_hbm.at[pl.ds(base + ci * cs, cs)])

    return comb


def kernel(x, Wg, W1, b1, W2, b2):
    bsz, seq, D = x.shape
    E, _, H = W1.shape
    T = bsz * seq
    K = 2
    cap = int(round(K * T * 1.05 / E))
    capp = -((-(cap + 2)) // 64) * 64
    nslot = E * capp
    nh = 8

    x2d = x.reshape(T, D)

    router = pl.pallas_call(
        functools.partial(_router_body, cap, capp),
        out_shape=(
            jax.ShapeDtypeStruct((T, 2), I32),
            jax.ShapeDtypeStruct((T, 2), I32),
            jax.ShapeDtypeStruct((T, 2), F32),
            jax.ShapeDtypeStruct((1, 1), F32),
            jax.ShapeDtypeStruct((1, 1), F32),
        ),
    )
    dest4, islot4, wval4, laux, lload = router(x2d, Wg)

    dest_sl = jnp.concatenate([dest4[:, 0], dest4[:, 1]])
    wval_sl = jnp.concatenate([wval4[:, 0], wval4[:, 1]])
    tok = jnp.arange(T, dtype=I32)
    tok_sl = jnp.concatenate([tok, tok])

    src, wslot = _make_scatter(nslot, K * T)(dest_sl, tok_sl, wval_sl)
    xd = _make_dispatch(nslot, D)(src, x2d)

    hb = H // nh
    ffn = pl.pallas_call(
        functools.partial(_ffn_body, nh),
        grid=(E, nh),
        in_specs=[
            pl.BlockSpec((1, capp, D), lambda e, h: (e, 0, 0)),
            pl.BlockSpec((1, D, hb), lambda e, h: (e, 0, h)),
            pl.BlockSpec((1, 1, hb), lambda e, h: (e, 0, h)),
            pl.BlockSpec((1, hb, D), lambda e, h: (e, h, 0)),
            pl.BlockSpec((1, 1, D), lambda e, h: (e, 0, 0)),
            pl.BlockSpec((1, capp, 1), lambda e, h: (e, 0, 0)),
        ],
        out_specs=pl.BlockSpec((1, capp, D), lambda e, h: (e, 0, 0)),
        out_shape=jax.ShapeDtypeStruct((E, capp, D), F32),
        scratch_shapes=[pltpu.VMEM((capp, D), BF16)],
        compiler_params=pltpu.CompilerParams(
            dimension_semantics=("arbitrary", "arbitrary")),
    )
    y = ffn(xd.reshape(E, capp, D), W1, b1.reshape(E, 1, H), W2,
            b2.reshape(E, 1, D), wslot.reshape(E, capp, 1))

    out2d = _make_combine(T, D)(y.reshape(nslot, D),
                                islot4[:, 0], islot4[:, 1])
    return (out2d.reshape(bsz, seq, D), laux[0, 0], lload[0, 0])


# speedup vs baseline: 1.7203x; 1.0835x over previous
import functools

import jax
import jax.numpy as jnp
from jax import lax
from jax.experimental import pallas as pl
from jax.experimental.pallas import tpu as pltpu
from jax.experimental.pallas import tpu_sc as plsc

F32 = jnp.float32
BF16 = jnp.bfloat16
I32 = jnp.int32

NUM_TILES = 32


def _tile_id():
    return lax.axis_index("s") * 2 + lax.axis_index("c")


def _router_body(cap, capp, x_ref, wg_ref, dest_ref, islot_ref,
                 wval_ref, laux_ref, lload_ref):
    T, D = x_ref.shape
    E = wg_ref.shape[0]
    CH = 1024
    x = x_ref[...]
    logits = lax.dot_general(x, wg_ref[...], (((1,), (1,)), ((), ())),
                             preferred_element_type=F32)
    m = jnp.max(logits, axis=1, keepdims=True)
    ex = jnp.exp(logits - m)
    gates = ex / jnp.sum(ex, axis=1, keepdims=True)

    it8 = lax.broadcasted_iota(I32, (T, E), 1)
    g1 = jnp.max(gates, axis=1, keepdims=True)
    i1 = jnp.min(jnp.where(gates == g1, it8, E), axis=1, keepdims=True)
    gm = jnp.where(it8 == i1, -jnp.inf, gates)
    g2 = jnp.max(gm, axis=1, keepdims=True)
    i2 = jnp.min(jnp.where(gm == g2, it8, E), axis=1, keepdims=True)

    imp = jnp.sum(gates, axis=0, keepdims=True)
    mi = jnp.mean(imp)
    si = jnp.sqrt(jnp.mean((imp - mi) ** 2))
    imp_loss = (si / (mi + 1e-6)) ** 2

    r = lax.broadcasted_iota(I32, (CH, CH), 0)
    c = lax.broadcasted_iota(I32, (CH, CH), 1)
    tri = (r >= c).astype(F32)
    ohA = (i1 == it8).astype(F32)
    ohB = (i2 == it8).astype(F32)
    carry = jnp.zeros((1, E), F32)
    pos = {0: [], 1: []}
    for key, oh in ((0, ohA), (1, ohB)):
        for ci in range(T // CH):
            blk = lax.slice(oh, (ci * CH, 0), ((ci + 1) * CH, E))
            cs = lax.dot_general(tri, blk, (((1,), (0,)), ((), ())),
                                 preferred_element_type=F32) + carry
            pos[key].append(jnp.sum(cs * blk, axis=1, keepdims=True) - 1.0)
            carry = carry + jnp.sum(blk, axis=0, keepdims=True)
    posA = jnp.concatenate(pos[0], axis=0)
    posB = jnp.concatenate(pos[1], axis=0)

    tpe = jnp.minimum(carry, float(cap))
    mt = jnp.mean(tpe)
    st = jnp.sqrt(jnp.mean((tpe - mt) ** 2))
    l_load = (st / (mt + 1e-6)) ** 2
    laux_ref[...] = jnp.reshape(0.5 * (imp_loss + l_load), (1, 1))
    lload_ref[...] = jnp.reshape(l_load, (1, 1))

    trash = cap
    dropg = cap + 1
    for i_, po, g_, col in ((i1, posA, g1, 0), (i2, posB, g2, 1)):
        pin = po.astype(I32)
        within = po < float(cap)
        slot = i_ * capp + pin
        dest_ref[:, col:col + 1] = jnp.where(within, slot, trash)
        islot_ref[:, col:col + 1] = jnp.where(within, slot, dropg)
        wval_ref[:, col:col + 1] = g_


def _make_scatter(nslot, n):
    mesh = plsc.VectorSubcoreMesh(core_axis_name="c", subcore_axis_name="s")

    @functools.partial(
        pl.kernel,
        out_type=(jax.ShapeDtypeStruct((nslot,), I32),
                  jax.ShapeDtypeStruct((nslot,), F32)),
        mesh=mesh,
        scratch_types=[
            pltpu.VMEM((n,), I32),
            pltpu.VMEM((n,), I32),
            pltpu.VMEM((n,), F32),
            pltpu.VMEM((nslot,), I32),
            pltpu.VMEM((nslot,), F32),
        ],
        compiler_params=pltpu.CompilerParams(needs_layout_passes=False),
    )
    def scat(dest_hbm, tok_hbm, wv_hbm, src_hbm, wslot_hbm,
             destv, tokv, wvv, srcv, wsv):
        @pl.when(_tile_id() == 0)
        def _():
            pltpu.sync_copy(dest_hbm, destv)
            pltpu.sync_copy(tok_hbm, tokv)
            pltpu.sync_copy(wv_hbm, wvv)
            zi = jnp.zeros((16,), I32)
            zf = jnp.zeros((16,), F32)

            def zb(i, _):
                srcv[pl.ds(i * 16, 16)] = zi
                wsv[pl.ds(i * 16, 16)] = zf
                return 0

            lax.fori_loop(0, nslot // 16, zb, 0)

            def sb(i, _):
                idx = destv[pl.ds(i * 16, 16)]
                plsc.store_scatter(srcv, [idx], tokv[pl.ds(i * 16, 16)])
                plsc.store_scatter(wsv, [idx], wvv[pl.ds(i * 16, 16)])
                return 0

            lax.fori_loop(0, n // 16, sb, 0)
            pltpu.sync_copy(srcv, src_hbm)
            pltpu.sync_copy(wsv, wslot_hbm)

    return scat


def _make_dispatch(nslot, D):
    per = nslot // NUM_TILES
    mesh = plsc.VectorSubcoreMesh(core_axis_name="c", subcore_axis_name="s")
    chunks = []
    off = 0
    while off < per:
        sz = min(32, per - off)
        chunks.append((off, sz))
        off += sz

    @functools.partial(
        pl.kernel,
        out_type=jax.ShapeDtypeStruct((nslot, D), F32),
        mesh=mesh,
        scratch_types=[
            pltpu.VMEM((per,), I32),
            pltpu.VMEM((32, D), F32),
            pltpu.VMEM((32, D), F32),
            pltpu.SemaphoreType.DMA,
            pltpu.SemaphoreType.DMA,
        ],
    )
    def disp(src_hbm, x_hbm, xd_hbm, idxv, rows0, rows1, sem0, sem1):
        base = _tile_id() * per
        pltpu.sync_copy(src_hbm.at[pl.ds(base, per)], idxv)
        bufs = (rows0, rows1)
        sems = (sem0, sem1)
        n = len(chunks)
        cps = [None] * n

        def issue(i):
            off, sz = chunks[i]
            cps[i] = pltpu.async_copy(x_hbm.at[idxv.at[pl.ds(off, sz)]],
                                      bufs[i % 2].at[pl.ds(0, sz)],
                                      sems[i % 2])

        issue(0)
        for i, (off, sz) in enumerate(chunks):
            if i + 1 < n:
                issue(i + 1)
            cps[i].wait()
            pltpu.sync_copy(bufs[i % 2].at[pl.ds(0, sz)],
                            xd_hbm.at[pl.ds(base + off, sz)])

    return disp


def _ffn_body(nh, xd_ref, w1_ref, b1_ref, w2_ref, b2_ref, wc_ref, y_ref,
              xbf):
    h = pl.program_id(1)

    @pl.when(h == 0)
    def _():
        xbf[...] = xd_ref[0].astype(BF16)
        y_ref[...] = jnp.zeros_like(y_ref)

    a = lax.dot_general(xbf[...], w1_ref[0].astype(BF16),
                        (((1,), (0,)), ((), ())), preferred_element_type=F32)
    a = a + b1_ref[0]
    g = 0.5 * a * (1.0 + lax.erf(a * 0.7071067811865476))
    acc = lax.dot_general(g.astype(BF16), w2_ref[0].astype(BF16),
                          (((1,), (0,)), ((), ())), preferred_element_type=F32)
    ynew = y_ref[0] + acc

    @pl.when(h < nh - 1)
    def _():
        y_ref[0] = ynew

    @pl.when(h == nh - 1)
    def _():
        y_ref[0] = (ynew + b2_ref[0]) * wc_ref[0]


def _make_combine(T, D):
    per = T // NUM_TILES
    cs = 8
    nch = per // cs
    mesh = plsc.VectorSubcoreMesh(core_axis_name="c", subcore_axis_name="s")

    @functools.partial(
        pl.kernel,
        out_type=jax.ShapeDtypeStruct((T, D), F32),
        mesh=mesh,
        scratch_types=[
            pltpu.VMEM((per,), I32),
            pltpu.VMEM((per,), I32),
            pltpu.VMEM((cs, D), F32),
            pltpu.VMEM((cs, D), F32),
            pltpu.VMEM((cs, D), F32),
            pltpu.VMEM((cs, D), F32),
            pltpu.SemaphoreType.DMA,
            pltpu.SemaphoreType.DMA,
            pltpu.SemaphoreType.DMA,
            pltpu.SemaphoreType.DMA,
        ],
    )
    def comb(ys_hbm, i0_hbm, i1_hbm, out_hbm, i0v, i1v,
             a0, a1, c0, c1, sa0, sa1, sc0, sc1):
        base = _tile_id() * per
        pltpu.sync_copy(i0_hbm.at[pl.ds(base, per)], i0v)
        pltpu.sync_copy(i1_hbm.at[pl.ds(base, per)], i1v)
        ngrp = D // 16
        abufs = (a0, a1)
        cbufs = (c0, c1)
        asems = (sa0, sa1)
        csems = (sc0, sc1)
        cpa = [None] * nch
        cpc = [None] * nch

        def issue(i):
            off = i * cs
            cpa[i] = pltpu.async_copy(ys_hbm.at[i0v.at[pl.ds(off, cs)]],
                                      abufs[i % 2], asems[i % 2])
            cpc[i] = pltpu.async_copy(ys_hbm.at[i1v.at[pl.ds(off, cs)]],
                                      cbufs[i % 2], csems[i % 2])

        issue(0)
        for ci in range(nch):
            if ci + 1 < nch:
                issue(ci + 1)
            cpa[ci].wait()
            cpc[ci].wait()
            aa = abufs[ci % 2]
            cc = cbufs[ci % 2]

            def add_row(rr, _):
                for u in range(ngrp):
                    aa[rr, pl.ds(u * 16, 16)] = (
                        aa[rr, pl.ds(u * 16, 16)] + cc[rr, pl.ds(u * 16, 16)])
                return 0

            lax.fori_loop(0, cs, add_row, 0)
            pltpu.sync_copy(aa, out_hbm.at[pl.ds(base + ci * cs, cs)])

    return comb


def kernel(x, Wg, W1, b1, W2, b2):
    bsz, seq, D = x.shape
    E, _, H = W1.shape
    T = bsz * seq
    K = 2
    cap = int(round(K * T * 1.05 / E))
    capp = -((-(cap + 2)) // 64) * 64
    nslot = E * capp
    nh = 4

    x2d = x.reshape(T, D)

    router = pl.pallas_call(
        functools.partial(_router_body, cap, capp),
        out_shape=(
            jax.ShapeDtypeStruct((T, 2), I32),
            jax.ShapeDtypeStruct((T, 2), I32),
            jax.ShapeDtypeStruct((T, 2), F32),
            jax.ShapeDtypeStruct((1, 1), F32),
            jax.ShapeDtypeStruct((1, 1), F32),
        ),
    )
    dest4, islot4, wval4, laux, lload = router(x2d, Wg)

    dest_sl = jnp.concatenate([dest4[:, 0], dest4[:, 1]])
    wval_sl = jnp.concatenate([wval4[:, 0], wval4[:, 1]])
    tok = jnp.arange(T, dtype=I32)
    tok_sl = jnp.concatenate([tok, tok])

    src, wslot = _make_scatter(nslot, K * T)(dest_sl, tok_sl, wval_sl)
    xd = _make_dispatch(nslot, D)(src, x2d)

    hb = H // nh
    ffn = pl.pallas_call(
        functools.partial(_ffn_body, nh),
        grid=(E, nh),
        in_specs=[
            pl.BlockSpec((1, capp, D), lambda e, h: (e, 0, 0)),
            pl.BlockSpec((1, D, hb), lambda e, h: (e, 0, h)),
            pl.BlockSpec((1, 1, hb), lambda e, h: (e, 0, h)),
            pl.BlockSpec((1, hb, D), lambda e, h: (e, h, 0)),
            pl.BlockSpec((1, 1, D), lambda e, h: (e, 0, 0)),
            pl.BlockSpec((1, capp, 1), lambda e, h: (e, 0, 0)),
        ],
        out_specs=pl.BlockSpec((1, capp, D), lambda e, h: (e, 0, 0)),
        out_shape=jax.ShapeDtypeStruct((E, capp, D), F32),
        scratch_shapes=[pltpu.VMEM((capp, D), BF16)],
        compiler_params=pltpu.CompilerParams(
            dimension_semantics=("arbitrary", "arbitrary")),
    )
    y = ffn(xd.reshape(E, capp, D), W1, b1.reshape(E, 1, H), W2,
            b2.reshape(E, 1, D), wslot.reshape(E, capp, 1))

    out2d = _make_combine(T, D)(y.reshape(nslot, D),
                                islot4[:, 0], islot4[:, 1])
    return (out2d.reshape(bsz, seq, D), laux[0, 0], lload[0, 0])


# speedup vs baseline: 1.7570x; 1.0213x over previous
import functools

import jax
import jax.numpy as jnp
from jax import lax
from jax.experimental import pallas as pl
from jax.experimental.pallas import tpu as pltpu
from jax.experimental.pallas import tpu_sc as plsc

F32 = jnp.float32
BF16 = jnp.bfloat16
I32 = jnp.int32

NUM_TILES = 32


def _tile_id():
    return lax.axis_index("s") * 2 + lax.axis_index("c")


def _router_body(cap, capp, x_ref, wg_ref, dest_ref, islot_ref,
                 wval_ref, laux_ref, lload_ref):
    T, D = x_ref.shape
    E = wg_ref.shape[0]
    CH = 1024
    x = x_ref[...]
    logits = lax.dot_general(x, wg_ref[...], (((1,), (1,)), ((), ())),
                             preferred_element_type=F32)
    m = jnp.max(logits, axis=1, keepdims=True)
    ex = jnp.exp(logits - m)
    gates = ex / jnp.sum(ex, axis=1, keepdims=True)

    it8 = lax.broadcasted_iota(I32, (T, E), 1)
    g1 = jnp.max(gates, axis=1, keepdims=True)
    i1 = jnp.min(jnp.where(gates == g1, it8, E), axis=1, keepdims=True)
    gm = jnp.where(it8 == i1, -jnp.inf, gates)
    g2 = jnp.max(gm, axis=1, keepdims=True)
    i2 = jnp.min(jnp.where(gm == g2, it8, E), axis=1, keepdims=True)

    imp = jnp.sum(gates, axis=0, keepdims=True)
    mi = jnp.mean(imp)
    si = jnp.sqrt(jnp.mean((imp - mi) ** 2))
    imp_loss = (si / (mi + 1e-6)) ** 2

    r = lax.broadcasted_iota(I32, (CH, CH), 0)
    c = lax.broadcasted_iota(I32, (CH, CH), 1)
    tri = (r >= c).astype(F32)
    ohA = (i1 == it8).astype(F32)
    ohB = (i2 == it8).astype(F32)
    carry = jnp.zeros((1, E), F32)
    pos = {0: [], 1: []}
    for key, oh in ((0, ohA), (1, ohB)):
        for ci in range(T // CH):
            blk = lax.slice(oh, (ci * CH, 0), ((ci + 1) * CH, E))
            cs = lax.dot_general(tri, blk, (((1,), (0,)), ((), ())),
                                 preferred_element_type=F32) + carry
            pos[key].append(jnp.sum(cs * blk, axis=1, keepdims=True) - 1.0)
            carry = carry + jnp.sum(blk, axis=0, keepdims=True)
    posA = jnp.concatenate(pos[0], axis=0)
    posB = jnp.concatenate(pos[1], axis=0)

    tpe = jnp.minimum(carry, float(cap))
    mt = jnp.mean(tpe)
    st = jnp.sqrt(jnp.mean((tpe - mt) ** 2))
    l_load = (st / (mt + 1e-6)) ** 2
    laux_ref[...] = jnp.reshape(0.5 * (imp_loss + l_load), (1, 1))
    lload_ref[...] = jnp.reshape(l_load, (1, 1))

    trash = cap
    dropg = cap + 1
    for i_, po, g_, col in ((i1, posA, g1, 0), (i2, posB, g2, 1)):
        pin = po.astype(I32)
        within = po < float(cap)
        slot = i_ * capp + pin
        dest_ref[:, col:col + 1] = jnp.where(within, slot, trash)
        islot_ref[:, col:col + 1] = jnp.where(within, slot, dropg)
        wval_ref[:, col:col + 1] = g_


def _make_scatter(nslot, n):
    mesh = plsc.VectorSubcoreMesh(core_axis_name="c", subcore_axis_name="s")

    @functools.partial(
        pl.kernel,
        out_type=(jax.ShapeDtypeStruct((nslot,), I32),
                  jax.ShapeDtypeStruct((nslot,), F32)),
        mesh=mesh,
        scratch_types=[
            pltpu.VMEM((n,), I32),
            pltpu.VMEM((n,), I32),
            pltpu.VMEM((n,), F32),
            pltpu.VMEM((nslot,), I32),
            pltpu.VMEM((nslot,), F32),
        ],
        compiler_params=pltpu.CompilerParams(needs_layout_passes=False),
    )
    def scat(dest_hbm, tok_hbm, wv_hbm, src_hbm, wslot_hbm,
             destv, tokv, wvv, srcv, wsv):
        @pl.when(_tile_id() == 0)
        def _():
            pltpu.sync_copy(dest_hbm, destv)
            pltpu.sync_copy(tok_hbm, tokv)
            pltpu.sync_copy(wv_hbm, wvv)
            zi = jnp.zeros((16,), I32)
            zf = jnp.zeros((16,), F32)

            def zb(i, _):
                srcv[pl.ds(i * 16, 16)] = zi
                wsv[pl.ds(i * 16, 16)] = zf
                return 0

            lax.fori_loop(0, nslot // 16, zb, 0)

            def sb(i, _):
                idx = destv[pl.ds(i * 16, 16)]
                plsc.store_scatter(srcv, [idx], tokv[pl.ds(i * 16, 16)])
                plsc.store_scatter(wsv, [idx], wvv[pl.ds(i * 16, 16)])
                return 0

            lax.fori_loop(0, n // 16, sb, 0)
            pltpu.sync_copy(srcv, src_hbm)
            pltpu.sync_copy(wsv, wslot_hbm)

    return scat


def _make_dispatch(nslot, D):
    per = nslot // NUM_TILES
    mesh = plsc.VectorSubcoreMesh(core_axis_name="c", subcore_axis_name="s")
    chunks = []
    off = 0
    while off < per:
        sz = min(32, per - off)
        chunks.append((off, sz))
        off += sz

    @functools.partial(
        pl.kernel,
        out_type=jax.ShapeDtypeStruct((nslot, D), F32),
        mesh=mesh,
        scratch_types=[
            pltpu.VMEM((per,), I32),
            pltpu.VMEM((32, D), F32),
            pltpu.VMEM((32, D), F32),
            pltpu.SemaphoreType.DMA,
            pltpu.SemaphoreType.DMA,
        ],
    )
    def disp(src_hbm, x_hbm, xd_hbm, idxv, rows0, rows1, sem0, sem1):
        base = _tile_id() * per
        pltpu.sync_copy(src_hbm.at[pl.ds(base, per)], idxv)
        bufs = (rows0, rows1)
        sems = (sem0, sem1)
        n = len(chunks)
        cps = [None] * n

        def issue(i):
            off, sz = chunks[i]
            cps[i] = pltpu.async_copy(x_hbm.at[idxv.at[pl.ds(off, sz)]],
                                      bufs[i % 2].at[pl.ds(0, sz)],
                                      sems[i % 2])

        issue(0)
        for i, (off, sz) in enumerate(chunks):
            if i + 1 < n:
                issue(i + 1)
            cps[i].wait()
            pltpu.sync_copy(bufs[i % 2].at[pl.ds(0, sz)],
                            xd_hbm.at[pl.ds(base + off, sz)])

    return disp


def _ffn_body(nh, xd_ref, w1_ref, b1_ref, w2_ref, b2_ref, wc_ref, y_ref,
              xbf):
    h = pl.program_id(1)

    @pl.when(h == 0)
    def _():
        xbf[...] = xd_ref[0].astype(BF16)
        y_ref[...] = jnp.zeros_like(y_ref)

    a = lax.dot_general(xbf[...], w1_ref[0].astype(BF16),
                        (((1,), (0,)), ((), ())), preferred_element_type=F32)
    a = a + b1_ref[0]
    g = 0.5 * a * (1.0 + lax.erf(a * 0.7071067811865476))
    acc = lax.dot_general(g.astype(BF16), w2_ref[0].astype(BF16),
                          (((1,), (0,)), ((), ())), preferred_element_type=F32)
    ynew = y_ref[0] + acc

    @pl.when(h < nh - 1)
    def _():
        y_ref[0] = ynew

    @pl.when(h == nh - 1)
    def _():
        y_ref[0] = (ynew + b2_ref[0]) * wc_ref[0]


def _make_combine(T, D):
    per = T // NUM_TILES
    cs = 8
    nch = per // cs
    mesh = plsc.VectorSubcoreMesh(core_axis_name="c", subcore_axis_name="s")

    @functools.partial(
        pl.kernel,
        out_type=jax.ShapeDtypeStruct((T, D), F32),
        mesh=mesh,
        scratch_types=[
            pltpu.VMEM((per,), I32),
            pltpu.VMEM((per,), I32),
            pltpu.VMEM((cs, D), F32),
            pltpu.VMEM((cs, D), F32),
            pltpu.VMEM((cs, D), F32),
            pltpu.VMEM((cs, D), F32),
            pltpu.SemaphoreType.DMA,
            pltpu.SemaphoreType.DMA,
            pltpu.SemaphoreType.DMA,
            pltpu.SemaphoreType.DMA,
        ],
    )
    def comb(ys_hbm, i0_hbm, i1_hbm, out_hbm, i0v, i1v,
             a0, a1, c0, c1, sa0, sa1, sc0, sc1):
        base = _tile_id() * per
        pltpu.sync_copy(i0_hbm.at[pl.ds(base, per)], i0v)
        pltpu.sync_copy(i1_hbm.at[pl.ds(base, per)], i1v)
        ngrp = D // 16
        abufs = (a0, a1)
        cbufs = (c0, c1)
        asems = (sa0, sa1)
        csems = (sc0, sc1)
        cpa = [None] * nch
        cpc = [None] * nch

        def issue(i):
            off = i * cs
            cpa[i] = pltpu.async_copy(ys_hbm.at[i0v.at[pl.ds(off, cs)]],
                                      abufs[i % 2], asems[i % 2])
            cpc[i] = pltpu.async_copy(ys_hbm.at[i1v.at[pl.ds(off, cs)]],
                                      cbufs[i % 2], csems[i % 2])

        issue(0)
        for ci in range(nch):
            if ci + 1 < nch:
                issue(ci + 1)
            cpa[ci].wait()
            cpc[ci].wait()
            aa = abufs[ci % 2]
            cc = cbufs[ci % 2]

            def add_row(rr, _):
                for u in range(ngrp):
                    aa[rr, pl.ds(u * 16, 16)] = (
                        aa[rr, pl.ds(u * 16, 16)] + cc[rr, pl.ds(u * 16, 16)])
                return 0

            lax.fori_loop(0, cs, add_row, 0)
            pltpu.sync_copy(aa, out_hbm.at[pl.ds(base + ci * cs, cs)])

    return comb


def kernel(x, Wg, W1, b1, W2, b2):
    bsz, seq, D = x.shape
    E, _, H = W1.shape
    T = bsz * seq
    K = 2
    cap = int(round(K * T * 1.05 / E))
    capp = -((-(cap + 2)) // 64) * 64
    nslot = E * capp
    nh = 2

    x2d = x.reshape(T, D)

    router = pl.pallas_call(
        functools.partial(_router_body, cap, capp),
        out_shape=(
            jax.ShapeDtypeStruct((T, 2), I32),
            jax.ShapeDtypeStruct((T, 2), I32),
            jax.ShapeDtypeStruct((T, 2), F32),
            jax.ShapeDtypeStruct((1, 1), F32),
            jax.ShapeDtypeStruct((1, 1), F32),
        ),
    )
    dest4, islot4, wval4, laux, lload = router(x2d, Wg)

    dest_sl = jnp.concatenate([dest4[:, 0], dest4[:, 1]])
    wval_sl = jnp.concatenate([wval4[:, 0], wval4[:, 1]])
    tok = jnp.arange(T, dtype=I32)
    tok_sl = jnp.concatenate([tok, tok])

    src, wslot = _make_scatter(nslot, K * T)(dest_sl, tok_sl, wval_sl)
    xd = _make_dispatch(nslot, D)(src, x2d)

    hb = H // nh
    ffn = pl.pallas_call(
        functools.partial(_ffn_body, nh),
        grid=(E, nh),
        in_specs=[
            pl.BlockSpec((1, capp, D), lambda e, h: (e, 0, 0)),
            pl.BlockSpec((1, D, hb), lambda e, h: (e, 0, h)),
            pl.BlockSpec((1, 1, hb), lambda e, h: (e, 0, h)),
            pl.BlockSpec((1, hb, D), lambda e, h: (e, h, 0)),
            pl.BlockSpec((1, 1, D), lambda e, h: (e, 0, 0)),
            pl.BlockSpec((1, capp, 1), lambda e, h: (e, 0, 0)),
        ],
        out_specs=pl.BlockSpec((1, capp, D), lambda e, h: (e, 0, 0)),
        out_shape=jax.ShapeDtypeStruct((E, capp, D), F32),
        scratch_shapes=[pltpu.VMEM((capp, D), BF16)],
        compiler_params=pltpu.CompilerParams(
            dimension_semantics=("arbitrary", "arbitrary"),
            vmem_limit_bytes=112 * 1024 * 1024),
    )
    y = ffn(xd.reshape(E, capp, D), W1, b1.reshape(E, 1, H), W2,
            b2.reshape(E, 1, D), wslot.reshape(E, capp, 1))

    out2d = _make_combine(T, D)(y.reshape(nslot, D),
                                islot4[:, 0], islot4[:, 1])
    return (out2d.reshape(bsz, seq, D), laux[0, 0], lload[0, 0])
